# baseline (speedup 1.0000x reference)
import jax
import jax.numpy as jnp
from jax import lax
from jax.experimental import pallas as pl
from jax.experimental.pallas import tpu as pltpu
from jax.experimental.pallas import tpu_sc as plsc

NSITES = 64
NTIMES = 8192
NBASE = 2016
NFRAMES = 4096

_PI = 3.141592653589793
_TWO_PI = 6.283185307179586

L = 16
NC = 2
NS = 16
NW = NC * NS
FPW = NFRAMES // NW
SG = 4
NSG = FPW // SG
ROW = 2 * NBASE
CVECS = NBASE // L
GPAD = 128


def _wrap(x):
    r = lax.rem(x + _PI, _TWO_PI)
    r = jnp.where(r < 0.0, r + _TWO_PI, r)
    return r - _PI


def _sc_body(bl_hbm, frames_hbm, gt_hbm, gi_hbm, gj_hbm,
             fidx_v, g2_v, gflat_v, rb0, rb1, oi0, oi1, oj0, oj1,
             sem_g, sr0, sr1, so0, so1):
    wid = lax.axis_index("s") * NC + lax.axis_index("c")
    base = wid * FPW
    iota = lax.iota(jnp.int32, L)
    perm = lax.bitwise_and(iota * 2, L - 1)
    permj = perm + 1
    lower = iota < (L // 2)

    pltpu.sync_copy(frames_hbm.at[pl.ds(base, FPW)], fidx_v.at[pl.ds(0, FPW)])
    pltpu.async_copy(
        gt_hbm.at[fidx_v.at[pl.ds(0, FPW)]], g2_v, sem_g).wait()

    def clip_body(k, carry):
        r = lax.shift_right_logical(k, 2)
        c = lax.bitwise_and(k, 3) * L
        gflat_v[pl.ds(k * L, L)] = _wrap(g2_v[r, pl.ds(c, L)])
        return carry

    lax.fori_loop(0, FPW * NSITES // L, clip_body, 0)

    rbufs = (rb0, rb1)
    oibufs = (oi0, oi1)
    ojbufs = (oj0, oj1)
    rsems = (sr0, sr1)
    osems = (so0, so1)

    def fire_rows(qq, h):
        tv = fidx_v[pl.ds(qq * SG, L)]
        for j in range(SG):
            t = tv[j]
            pltpu.async_copy(
                bl_hbm.at[pl.ds(t * ROW, ROW)],
                rbufs[h].at[pl.ds(j * ROW, ROW)], rsems[h])

    def compute(qq, h):
        rb, oi, oj = rbufs[h], oibufs[h], ojbufs[h]
        for j in range(SG):
            fofs = jnp.full((L,), (qq * SG + j) * NSITES, jnp.int32)
            rbase = j * ROW
            obase = j * NBASE

            def c_body(c, carry):
                a = rb[pl.ds(rbase + c * 2 * L, L)]
                b = rb[pl.ds(rbase + c * 2 * L + L, L)]
                va = plsc.load_gather(gflat_v, [a + fofs])
                vb = plsc.load_gather(gflat_v, [b + fofs])
                gia = va.at[perm].get(mode="promise_in_bounds")
                gib = vb.at[perm].get(mode="promise_in_bounds")
                gja = va.at[permj].get(mode="promise_in_bounds")
                gjb = vb.at[permj].get(mode="promise_in_bounds")
                oi[pl.ds(obase + c * L, L)] = jnp.where(lower, gia, gib)
                oj[pl.ds(obase + c * L, L)] = jnp.where(lower, gja, gjb)
                return carry

            lax.fori_loop(0, CVECS, c_body, 0)

    fire_rows(0, 0)

    def pair_body(p, carry):
        for h in range(2):
            qq = 2 * p + h

            @pl.when(qq < NSG - 1)
            def _():
                fire_rows(qq + 1, (h + 1) % 2)

            @pl.when(qq >= 2)
            def _():
                off2 = (base + (qq - 2) * SG) * NBASE
                pltpu.make_async_copy(
                    oibufs[h], gi_hbm.at[pl.ds(off2, SG * NBASE)],
                    osems[h]).wait()
                pltpu.make_async_copy(
                    ojbufs[h], gj_hbm.at[pl.ds(off2, SG * NBASE)],
                    osems[h]).wait()

            pltpu.make_async_copy(
                bl_hbm.at[pl.ds(0, SG * ROW)], rbufs[h], rsems[h]).wait()

            compute(qq, h)

            off = (base + qq * SG) * NBASE
            pltpu.async_copy(
                oibufs[h], gi_hbm.at[pl.ds(off, SG * NBASE)], osems[h])
            pltpu.async_copy(
                ojbufs[h], gj_hbm.at[pl.ds(off, SG * NBASE)], osems[h])
        return carry

    lax.fori_loop(0, NSG // 2, pair_body, 0)

    for h in range(2):
        qq = NSG - 2 + h
        off2 = (base + qq * SG) * NBASE
        pltpu.make_async_copy(
            oibufs[h], gi_hbm.at[pl.ds(off2, SG * NBASE)], osems[h]).wait()
        pltpu.make_async_copy(
            ojbufs[h], gj_hbm.at[pl.ds(off2, SG * NBASE)], osems[h]).wait()


def _phase_gains_sc(bl_flat, frames, gt):
    k = pl.kernel(
        _sc_body,
        out_type=[
            jax.ShapeDtypeStruct((NFRAMES * NBASE,), jnp.float32),
            jax.ShapeDtypeStruct((NFRAMES * NBASE,), jnp.float32),
        ],
        mesh=plsc.VectorSubcoreMesh(core_axis_name="c", subcore_axis_name="s"),
        scratch_types=[
            pltpu.VMEM((FPW + L,), jnp.int32),
            pltpu.VMEM((FPW, GPAD), jnp.float32),
            pltpu.VMEM((FPW * NSITES,), jnp.float32),
            pltpu.VMEM((SG * ROW,), jnp.int32),
            pltpu.VMEM((SG * ROW,), jnp.int32),
            pltpu.VMEM((SG * NBASE,), jnp.float32),
            pltpu.VMEM((SG * NBASE,), jnp.float32),
            pltpu.VMEM((SG * NBASE,), jnp.float32),
            pltpu.VMEM((SG * NBASE,), jnp.float32),
            pltpu.SemaphoreType.DMA,
            pltpu.SemaphoreType.DMA,
            pltpu.SemaphoreType.DMA,
            pltpu.SemaphoreType.DMA,
            pltpu.SemaphoreType.DMA,
        ],
        compiler_params=pltpu.CompilerParams(needs_layout_passes=False),
    )
    return k(bl_flat, frames, gt)


def kernel(baselines, frames, gains):
    bl_flat = baselines.reshape(NTIMES * ROW)
    gt = jnp.pad(gains.T, ((0, 0), (0, GPAD - NSITES)))
    gi, gj = _phase_gains_sc(bl_flat, frames, gt)
    return gi.reshape(NFRAMES, NBASE), gj.reshape(NFRAMES, NBASE)


# speedup vs baseline: 29.7115x; 29.7115x over previous
import jax
import jax.numpy as jnp
from jax import lax
from jax.experimental import pallas as pl
from jax.experimental.pallas import tpu as pltpu
from jax.experimental.pallas import tpu_sc as plsc

NSITES = 64
NTIMES = 8192
NBASE = 2016
NFRAMES = 4096

_PI = 3.141592653589793
_TWO_PI = 6.283185307179586

L = 16
NC = 2
NS = 16
NW = NC * NS
FPW = NFRAMES // NW
SG = 4
NSG = FPW // SG
ROW = 2 * NBASE
CVECS = NBASE // L
GPAD = 128


def _wrap(x):
    r = lax.rem(x + _PI, _TWO_PI)
    r = jnp.where(r < 0.0, r + _TWO_PI, r)
    return r - _PI


def _sc_body(bl_hbm, frames_hbm, gt_hbm, gi_hbm, gj_hbm,
             fidx_v, g2_v, gflat_v, rb0, rb1, oi0, oi1, oj0, oj1,
             sem_g, sr0, sr1, so0, so1):
    wid = lax.axis_index("s") * NC + lax.axis_index("c")
    base = wid * FPW
    iota = lax.iota(jnp.int32, L)
    perm = lax.bitwise_and(iota * 2, L - 1)
    permj = perm + 1
    lower = iota < (L // 2)

    pltpu.sync_copy(frames_hbm.at[pl.ds(base, FPW)], fidx_v.at[pl.ds(0, FPW)])
    pltpu.async_copy(
        gt_hbm.at[fidx_v.at[pl.ds(0, FPW)]], g2_v, sem_g).wait()

    def clip_body(k, carry):
        r = lax.shift_right_logical(k, 2)
        c = lax.bitwise_and(k, 3) * L
        gflat_v[pl.ds(k * L, L)] = _wrap(g2_v[r, pl.ds(c, L)])
        return carry

    lax.fori_loop(0, FPW * NSITES // L, clip_body, 0)

    rbufs = (rb0, rb1)
    oibufs = (oi0, oi1)
    ojbufs = (oj0, oj1)
    rsems = (sr0, sr1)
    osems = (so0, so1)

    def fire_rows(qq, h):
        tv = fidx_v[pl.ds(qq * SG, L)]
        for j in range(SG):
            t = tv[j]
            pltpu.async_copy(
                bl_hbm.at[t], rbufs[h].at[pl.ds(j * ROW, ROW)], rsems[h])

    def compute(qq, h):
        rb, oi, oj = rbufs[h], oibufs[h], ojbufs[h]
        for j in range(SG):
            fofs = jnp.full((L,), (qq * SG + j) * NSITES, jnp.int32)
            rbase = j * ROW
            obase = j * NBASE

            def c_body(c, carry):
                a = rb[pl.ds(rbase + c * 2 * L, L)]
                b = rb[pl.ds(rbase + c * 2 * L + L, L)]
                va = plsc.load_gather(gflat_v, [a + fofs])
                vb = plsc.load_gather(gflat_v, [b + fofs])
                gia = va.at[perm].get(mode="promise_in_bounds")
                gib = vb.at[perm].get(mode="promise_in_bounds")
                gja = va.at[permj].get(mode="promise_in_bounds")
                gjb = vb.at[permj].get(mode="promise_in_bounds")
                oi[pl.ds(obase + c * L, L)] = jnp.where(lower, gia, gib)
                oj[pl.ds(obase + c * L, L)] = jnp.where(lower, gja, gjb)
                return carry

            lax.fori_loop(0, CVECS, c_body, 0)

    fire_rows(0, 0)

    def pair_body(p, carry):
        for h in range(2):
            qq = 2 * p + h

            @pl.when(qq < NSG - 1)
            def _():
                fire_rows(qq + 1, (h + 1) % 2)

            @pl.when(qq >= 2)
            def _():
                off2 = (base + (qq - 2) * SG) * NBASE
                pltpu.make_async_copy(
                    oibufs[h], gi_hbm.at[pl.ds(off2, SG * NBASE)],
                    osems[h]).wait()
                pltpu.make_async_copy(
                    ojbufs[h], gj_hbm.at[pl.ds(off2, SG * NBASE)],
                    osems[h]).wait()

            for j in range(SG):
                pltpu.make_async_copy(
                    bl_hbm.at[0], rbufs[h].at[pl.ds(j * ROW, ROW)],
                    rsems[h]).wait()

            compute(qq, h)

            off = (base + qq * SG) * NBASE
            pltpu.async_copy(
                oibufs[h], gi_hbm.at[pl.ds(off, SG * NBASE)], osems[h])
            pltpu.async_copy(
                ojbufs[h], gj_hbm.at[pl.ds(off, SG * NBASE)], osems[h])
        return carry

    lax.fori_loop(0, NSG // 2, pair_body, 0)

    for h in range(2):
        qq = NSG - 2 + h
        off2 = (base + qq * SG) * NBASE
        pltpu.make_async_copy(
            oibufs[h], gi_hbm.at[pl.ds(off2, SG * NBASE)], osems[h]).wait()
        pltpu.make_async_copy(
            ojbufs[h], gj_hbm.at[pl.ds(off2, SG * NBASE)], osems[h]).wait()


def _phase_gains_sc(bl_flat, frames, gt):
    k = pl.kernel(
        _sc_body,
        out_type=[
            jax.ShapeDtypeStruct((NFRAMES * NBASE,), jnp.float32),
            jax.ShapeDtypeStruct((NFRAMES * NBASE,), jnp.float32),
        ],
        mesh=plsc.VectorSubcoreMesh(core_axis_name="c", subcore_axis_name="s"),
        scratch_types=[
            pltpu.VMEM((FPW + L,), jnp.int32),
            pltpu.VMEM((FPW, NSITES), jnp.float32),
            pltpu.VMEM((FPW * NSITES,), jnp.float32),
            pltpu.VMEM((SG * ROW,), jnp.int32),
            pltpu.VMEM((SG * ROW,), jnp.int32),
            pltpu.VMEM((SG * NBASE,), jnp.float32),
            pltpu.VMEM((SG * NBASE,), jnp.float32),
            pltpu.VMEM((SG * NBASE,), jnp.float32),
            pltpu.VMEM((SG * NBASE,), jnp.float32),
            pltpu.SemaphoreType.DMA,
            pltpu.SemaphoreType.DMA,
            pltpu.SemaphoreType.DMA,
            pltpu.SemaphoreType.DMA,
            pltpu.SemaphoreType.DMA,
        ],
        compiler_params=pltpu.CompilerParams(
            needs_layout_passes=False, use_tc_tiling_on_sc=False),
    )
    return k(bl_flat, frames, gt)


def kernel(baselines, frames, gains):
    bl2 = baselines.reshape(NTIMES, ROW)
    gt = gains.T
    gi, gj = _phase_gains_sc(bl2, frames, gt)
    return gi.reshape(NFRAMES, NBASE), gj.reshape(NFRAMES, NBASE)


# speedup vs baseline: 31.4064x; 1.0570x over previous
import jax
import jax.numpy as jnp
from jax import lax
from jax.experimental import pallas as pl
from jax.experimental.pallas import tpu as pltpu
from jax.experimental.pallas import tpu_sc as plsc

NSITES = 64
NTIMES = 8192
NBASE = 2016
NFRAMES = 4096

_PI = 3.141592653589793
_TWO_PI = 6.283185307179586

L = 16
NC = 2
NS = 16
NW = NC * NS
FPW = NFRAMES // NW
SG = 4
GF = 8
NG = FPW // GF
ROW = 2 * NBASE
RPAD = 4096
CVECS = NBASE // L
GPAD = 128


def _wrap(x):
    r = lax.rem(x + _PI, _TWO_PI)
    r = jnp.where(r < 0.0, r + _TWO_PI, r)
    return r - _PI


def _sc_body(bl_hbm, frames_hbm, gt_hbm, gi_hbm, gj_hbm,
             fidx_v, g2_v, gflat_v, rb0, rb1, oi0, oi1, oj0, oj1,
             sem_g, sr0, sr1, so0, so1):
    wid = lax.axis_index("s") * NC + lax.axis_index("c")
    base = wid * FPW
    iota = lax.iota(jnp.int32, L)
    perm = lax.bitwise_and(iota * 2, L - 1)
    permj = perm + 1
    lower = iota < (L // 2)

    pltpu.sync_copy(frames_hbm.at[pl.ds(base, FPW)], fidx_v.at[pl.ds(0, FPW)])
    pltpu.async_copy(
        gt_hbm.at[fidx_v.at[pl.ds(0, FPW)]], g2_v, sem_g).wait()

    def clip_body(k, carry):
        r = lax.shift_right_logical(k, 2)
        c = lax.bitwise_and(k, 3) * L
        gflat_v[pl.ds(k * L, L)] = _wrap(g2_v[r, pl.ds(c, L)])
        return carry

    lax.fori_loop(0, FPW * NSITES // L, clip_body, 0)

    rbufs = (rb0, rb1)
    oibufs = (oi0, oi1)
    ojbufs = (oj0, oj1)
    rsems = (sr0, sr1)
    osems = (so0, so1)

    def fire_rows(g, h):
        pltpu.async_copy(
            bl_hbm.at[fidx_v.at[pl.ds(g * GF, GF)]], rbufs[h], rsems[h])

    def compute(g, h, q):
        rb, oi, oj = rbufs[h], oibufs[q], ojbufs[q]
        for j in range(SG):
            jj = SG * q + j
            fofs = jnp.full((L,), (g * GF + jj) * NSITES, jnp.int32)
            obase = j * NBASE

            def c_body(c, carry):
                a = rb[jj, pl.ds(c * 2 * L, L)]
                b = rb[jj, pl.ds(c * 2 * L + L, L)]
                va = plsc.load_gather(gflat_v, [a + fofs])
                vb = plsc.load_gather(gflat_v, [b + fofs])
                gia = va.at[perm].get(mode="promise_in_bounds")
                gib = vb.at[perm].get(mode="promise_in_bounds")
                gja = va.at[permj].get(mode="promise_in_bounds")
                gjb = vb.at[permj].get(mode="promise_in_bounds")
                oi[pl.ds(obase + c * L, L)] = jnp.where(lower, gia, gib)
                oj[pl.ds(obase + c * L, L)] = jnp.where(lower, gja, gjb)
                return carry

            lax.fori_loop(0, CVECS, c_body, 0)

    fire_rows(0, 0)

    def pair_body(p, carry):
        for h in range(2):
            g = 2 * p + h

            @pl.when(g < NG - 1)
            def _():
                fire_rows(g + 1, (h + 1) % 2)

            pltpu.make_async_copy(
                bl_hbm.at[fidx_v.at[pl.ds(g * GF, GF)]], rbufs[h],
                rsems[h]).wait()

            for q in range(2):
                @pl.when(g >= 1)
                def _():
                    off2 = (base + (g - 1) * GF + q * SG) * NBASE
                    pltpu.make_async_copy(
                        oibufs[q], gi_hbm.at[pl.ds(off2, SG * NBASE)],
                        osems[q]).wait()
                    pltpu.make_async_copy(
                        ojbufs[q], gj_hbm.at[pl.ds(off2, SG * NBASE)],
                        osems[q]).wait()

                compute(g, h, q)

                off = (base + g * GF + q * SG) * NBASE
                pltpu.async_copy(
                    oibufs[q], gi_hbm.at[pl.ds(off, SG * NBASE)], osems[q])
                pltpu.async_copy(
                    ojbufs[q], gj_hbm.at[pl.ds(off, SG * NBASE)], osems[q])
        return carry

    lax.fori_loop(0, NG // 2, pair_body, 0)

    for q in range(2):
        off2 = (base + (NG - 1) * GF + q * SG) * NBASE
        pltpu.make_async_copy(
            oibufs[q], gi_hbm.at[pl.ds(off2, SG * NBASE)], osems[q]).wait()
        pltpu.make_async_copy(
            ojbufs[q], gj_hbm.at[pl.ds(off2, SG * NBASE)], osems[q]).wait()


def _phase_gains_sc(bl_flat, frames, gt):
    k = pl.kernel(
        _sc_body,
        out_type=[
            jax.ShapeDtypeStruct((NFRAMES * NBASE,), jnp.float32),
            jax.ShapeDtypeStruct((NFRAMES * NBASE,), jnp.float32),
        ],
        mesh=plsc.VectorSubcoreMesh(core_axis_name="c", subcore_axis_name="s"),
        scratch_types=[
            pltpu.VMEM((FPW + L,), jnp.int32),
            pltpu.VMEM((FPW, GPAD), jnp.float32),
            pltpu.VMEM((FPW * NSITES,), jnp.float32),
            pltpu.VMEM((GF, RPAD), jnp.int32),
            pltpu.VMEM((GF, RPAD), jnp.int32),
            pltpu.VMEM((SG * NBASE,), jnp.float32),
            pltpu.VMEM((SG * NBASE,), jnp.float32),
            pltpu.VMEM((SG * NBASE,), jnp.float32),
            pltpu.VMEM((SG * NBASE,), jnp.float32),
            pltpu.SemaphoreType.DMA,
            pltpu.SemaphoreType.DMA,
            pltpu.SemaphoreType.DMA,
            pltpu.SemaphoreType.DMA,
            pltpu.SemaphoreType.DMA,
        ],
        compiler_params=pltpu.CompilerParams(needs_layout_passes=False),
    )
    return k(bl_flat, frames, gt)


def kernel(baselines, frames, gains):
    bl2 = jnp.pad(baselines.reshape(NTIMES, ROW), ((0, 0), (0, RPAD - ROW)))
    gt = jnp.pad(gains.T, ((0, 0), (0, GPAD - NSITES)))
    gi, gj = _phase_gains_sc(bl2, frames, gt)
    return gi.reshape(NFRAMES, NBASE), gj.reshape(NFRAMES, NBASE)


# speedup vs baseline: 41.1557x; 1.3104x over previous
import jax
import jax.numpy as jnp
from jax import lax
from jax.experimental import pallas as pl
from jax.experimental.pallas import tpu as pltpu
from jax.experimental.pallas import tpu_sc as plsc

NSITES = 64
NTIMES = 8192
NBASE = 2016
NFRAMES = 4096

_PI = 3.141592653589793
_TWO_PI = 6.283185307179586

L = 16
NC = 2
NS = 16
NW = NC * NS
FPW = NFRAMES // NW
SG = 4
GF = 8
NG = FPW // GF
ROW = 2 * NBASE
RPAD = 4096
CVECS = NBASE // L
GPAD = 128


def _wrap(x):
    r = lax.rem(x + _PI, _TWO_PI)
    r = jnp.where(r < 0.0, r + _TWO_PI, r)
    return r - _PI


def _sc_body(bl_hbm, frames_hbm, gt_hbm, gi_hbm, gj_hbm,
             fidx_v, g2_v, gflat_v, rb0, rb1, oi0, oi1, oj0, oj1,
             sem_g, sr0, sr1, so0, so1):
    wid = lax.axis_index("s") * NC + lax.axis_index("c")
    base = wid * FPW
    iota = lax.iota(jnp.int32, L)
    perm = lax.bitwise_and(iota * 2, L - 1)
    permj = perm + 1
    lower = iota < (L // 2)

    pltpu.sync_copy(frames_hbm.at[pl.ds(base, FPW)], fidx_v.at[pl.ds(0, FPW)])
    pltpu.async_copy(
        gt_hbm.at[fidx_v.at[pl.ds(0, FPW)]], g2_v, sem_g).wait()

    @plsc.parallel_loop(0, FPW * NSITES // L, 1, unroll=4)
    def _(k):
        r = lax.shift_right_logical(k, 2)
        c = lax.bitwise_and(k, 3) * L
        gflat_v[pl.ds(k * L, L)] = _wrap(g2_v[r, pl.ds(c, L)])

    rbufs = (rb0, rb1)
    oibufs = (oi0, oi1)
    ojbufs = (oj0, oj1)
    rsems = (sr0, sr1)
    osems = (so0, so1)

    def fire_rows(g, h):
        pltpu.async_copy(
            bl_hbm.at[fidx_v.at[pl.ds(g * GF, GF)]], rbufs[h], rsems[h])

    def compute(g, h, q):
        rb, oi, oj = rbufs[h], oibufs[q], ojbufs[q]
        for j in range(SG):
            jj = SG * q + j
            fofs = jnp.full((L,), (g * GF + jj) * NSITES, jnp.int32)
            obase = j * NBASE

            @plsc.parallel_loop(0, CVECS, 1, unroll=4)
            def _(c):
                a = rb[jj, pl.ds(c * 2 * L, L)]
                b = rb[jj, pl.ds(c * 2 * L + L, L)]
                va = plsc.load_gather(gflat_v, [a + fofs])
                vb = plsc.load_gather(gflat_v, [b + fofs])
                gia = va.at[perm].get(mode="promise_in_bounds")
                gib = vb.at[perm].get(mode="promise_in_bounds")
                gja = va.at[permj].get(mode="promise_in_bounds")
                gjb = vb.at[permj].get(mode="promise_in_bounds")
                oi[pl.ds(obase + c * L, L)] = jnp.where(lower, gia, gib)
                oj[pl.ds(obase + c * L, L)] = jnp.where(lower, gja, gjb)

    fire_rows(0, 0)

    def pair_body(p, carry):
        for h in range(2):
            g = 2 * p + h

            @pl.when(g < NG - 1)
            def _():
                fire_rows(g + 1, (h + 1) % 2)

            pltpu.make_async_copy(
                bl_hbm.at[fidx_v.at[pl.ds(g * GF, GF)]], rbufs[h],
                rsems[h]).wait()

            for q in range(2):
                @pl.when(g >= 1)
                def _():
                    off2 = (base + (g - 1) * GF + q * SG) * NBASE
                    pltpu.make_async_copy(
                        oibufs[q], gi_hbm.at[pl.ds(off2, SG * NBASE)],
                        osems[q]).wait()
                    pltpu.make_async_copy(
                        ojbufs[q], gj_hbm.at[pl.ds(off2, SG * NBASE)],
                        osems[q]).wait()

                compute(g, h, q)

                off = (base + g * GF + q * SG) * NBASE
                pltpu.async_copy(
                    oibufs[q], gi_hbm.at[pl.ds(off, SG * NBASE)], osems[q])
                pltpu.async_copy(
                    ojbufs[q], gj_hbm.at[pl.ds(off, SG * NBASE)], osems[q])
        return carry

    lax.fori_loop(0, NG // 2, pair_body, 0)

    for q in range(2):
        off2 = (base + (NG - 1) * GF + q * SG) * NBASE
        pltpu.make_async_copy(
            oibufs[q], gi_hbm.at[pl.ds(off2, SG * NBASE)], osems[q]).wait()
        pltpu.make_async_copy(
            ojbufs[q], gj_hbm.at[pl.ds(off2, SG * NBASE)], osems[q]).wait()


def _phase_gains_sc(bl_flat, frames, gt):
    k = pl.kernel(
        _sc_body,
        out_type=[
            jax.ShapeDtypeStruct((NFRAMES * NBASE,), jnp.float32),
            jax.ShapeDtypeStruct((NFRAMES * NBASE,), jnp.float32),
        ],
        mesh=plsc.VectorSubcoreMesh(core_axis_name="c", subcore_axis_name="s"),
        scratch_types=[
            pltpu.VMEM((FPW + L,), jnp.int32),
            pltpu.VMEM((FPW, GPAD), jnp.float32),
            pltpu.VMEM((FPW * NSITES,), jnp.float32),
            pltpu.VMEM((GF, RPAD), jnp.int32),
            pltpu.VMEM((GF, RPAD), jnp.int32),
            pltpu.VMEM((SG * NBASE,), jnp.float32),
            pltpu.VMEM((SG * NBASE,), jnp.float32),
            pltpu.VMEM((SG * NBASE,), jnp.float32),
            pltpu.VMEM((SG * NBASE,), jnp.float32),
            pltpu.SemaphoreType.DMA,
            pltpu.SemaphoreType.DMA,
            pltpu.SemaphoreType.DMA,
            pltpu.SemaphoreType.DMA,
            pltpu.SemaphoreType.DMA,
        ],
        compiler_params=pltpu.CompilerParams(needs_layout_passes=False),
    )
    return k(bl_flat, frames, gt)


def kernel(baselines, frames, gains):
    bl2 = jnp.pad(baselines.reshape(NTIMES, ROW), ((0, 0), (0, RPAD - ROW)))
    gt = jnp.pad(gains.T, ((0, 0), (0, GPAD - NSITES)))
    gi, gj = _phase_gains_sc(bl2, frames, gt)
    return gi.reshape(NFRAMES, NBASE), gj.reshape(NFRAMES, NBASE)
